# Initial kernel scaffold; baseline (speedup 1.0000x reference)
import functools

import jax
import jax.numpy as jnp
from jax import lax
from jax.experimental import pallas as pl
from jax.experimental.pallas import tpu as pltpu
from jax.experimental.pallas import tpu_sc as plsc

B = 1024
S = 512
D = 256
L = 16
NC = 2
NS = 16
NW = NC * NS
SPW = S // NW
NB = 64
CPB = B // NB
NCH = SPW * CPB
NBUF = 4
DJ = D // L
EPS = 1e-12


def _full16(x, dtype=jnp.int32):
    return lax.broadcast_in_dim(jnp.asarray(x, dtype=dtype), (L,), ())


def _rsqrt16(v):
    iv = plsc.bitcast(v, jnp.int32)
    magic = _full16(0x5F3759DF)
    y = plsc.bitcast(magic - (iv >> 1), jnp.float32)
    half = v * jnp.float32(0.5)
    for _ in range(3):
        y = y * (jnp.float32(1.5) - half * y * y)
    return y


def _sc_body(ids_h, tt_h, word_h, pos_h, type_h, gam_h, bet_h, out_h,
             idx_v, ttv_v, pos_v, typ_v, gam_v, bet_v, pt0_v, rows_v,
             g0, g1, g2, g3, s0, s1, s2, s3):
    gsems = [g0, g1, g2, g3]
    ssems = [s0, s1, s2, s3]
    wid = lax.axis_index("s") * NC + lax.axis_index("c")
    p0 = wid * SPW

    pltpu.sync_copy(ids_h.at[pl.ds(p0, SPW)], idx_v)
    pltpu.sync_copy(tt_h.at[pl.ds(p0, SPW)], ttv_v)
    pltpu.sync_copy(pos_h.at[pl.ds(p0, SPW)], pos_v)
    pltpu.sync_copy(type_h, typ_v)
    pltpu.sync_copy(gam_h, gam_v)
    pltpu.sync_copy(bet_h, bet_v)

    t0 = [typ_v[0, pl.ds(L * j, L)] for j in range(DJ)]
    dt = [typ_v[1, pl.ds(L * j, L)] - t0[j] for j in range(DJ)]
    gam = [gam_v[pl.ds(L * j, L)] for j in range(DJ)]
    bet = [bet_v[pl.ds(L * j, L)] for j in range(DJ)]

    for p in range(SPW):
        for j in range(DJ):
            pt0_v[p, pl.ds(L * j, L)] = pos_v[p, pl.ds(L * j, L)] + t0[j]

    inv_d = jnp.float32(1.0 / D)

    def chunk_coords(gg):
        s_local = gg // CPB
        b0 = (gg % CPB) * NB
        return s_local, b0

    def gather_copy(gg, buf):
        s_local, b0 = chunk_coords(gg)
        return pltpu.make_async_copy(
            word_h.at[idx_v.at[s_local, pl.ds(b0, NB)]],
            rows_v.at[buf], gsems[buf])

    def scatter_copy(gg, buf):
        s_local, b0 = chunk_coords(gg)
        col0 = (p0 + s_local) * D
        return pltpu.make_async_copy(
            rows_v.at[buf], out_h.at[pl.ds(b0, NB), pl.ds(col0, D)],
            ssems[buf])

    gather_copy(0, 0).start()
    gather_copy(1, 1).start()

    @pl.loop(0, NCH // NBUF)
    def _outer(g):
        for ph in range(NBUF):
            gg = g * NBUF + ph

            @pl.when(gg >= 2)
            def _():
                scatter_copy(gg - 2, (ph + 2) % NBUF).wait()

            @pl.when(gg + 2 < NCH)
            def _():
                gather_copy(gg + 2, (ph + 2) % NBUF).start()

            gather_copy(gg, ph).wait()

            s_local, b0 = chunk_coords(gg)
            prow = [pt0_v[s_local, pl.ds(L * j, L)] for j in range(DJ)]
            rbuf = rows_v.at[ph]

            @pl.loop(0, NB)
            def _tok(t):
                tsp = plsc.load_gather(
                    ttv_v, [_full16(s_local), _full16(b0 + t)])
                ttf = tsp.astype(jnp.float32)
                x = []
                for j in range(DJ):
                    w = rbuf[t, pl.ds(L * j, L)]
                    x.append(w + prow[j] + ttf * dt[j])
                acc = x[0]
                for j in range(1, DJ):
                    acc = acc + x[j]
                acc2 = x[0] * x[0]
                for j in range(1, DJ):
                    acc2 = acc2 + x[j] * x[j]
                mu = jnp.sum(acc) * inv_d
                m2 = jnp.sum(acc2) * inv_d
                var = m2 - mu * mu
                r = _rsqrt16(_full16(var + jnp.float32(EPS), jnp.float32))
                muv = _full16(mu, jnp.float32)
                for j in range(DJ):
                    rbuf[t, pl.ds(L * j, L)] = (
                        (x[j] - muv) * r * gam[j] + bet[j])

            scatter_copy(gg, ph).start()

    scatter_copy(NCH - 2, (NCH - 2) % NBUF).wait()
    scatter_copy(NCH - 1, (NCH - 1) % NBUF).wait()


@functools.partial(
    pl.kernel,
    out_type=jax.ShapeDtypeStruct((B, S * D), jnp.float32),
    mesh=plsc.VectorSubcoreMesh(
        core_axis_name="c", subcore_axis_name="s",
        num_cores=NC, num_subcores=NS),
    scratch_types=[
        pltpu.VMEM((SPW, B), jnp.int32),
        pltpu.VMEM((SPW, B), jnp.int32),
        pltpu.VMEM((SPW, D), jnp.float32),
        pltpu.VMEM((2, D), jnp.float32),
        pltpu.VMEM((D,), jnp.float32),
        pltpu.VMEM((D,), jnp.float32),
        pltpu.VMEM((SPW, D), jnp.float32),
        pltpu.VMEM((NBUF, NB, D), jnp.float32),
        pltpu.SemaphoreType.DMA,
        pltpu.SemaphoreType.DMA,
        pltpu.SemaphoreType.DMA,
        pltpu.SemaphoreType.DMA,
        pltpu.SemaphoreType.DMA,
        pltpu.SemaphoreType.DMA,
        pltpu.SemaphoreType.DMA,
        pltpu.SemaphoreType.DMA,
    ],
)
def _bert_emb_sc(ids_h, tt_h, word_h, pos_h, type_h, gam_h, bet_h, out_h,
                 *scratch):
    _sc_body(ids_h, tt_h, word_h, pos_h, type_h, gam_h, bet_h, out_h,
             *scratch)


def kernel(input_ids, token_type_ids, word_emb, pos_emb, type_emb,
           ln_gamma, ln_beta):
    ids_t = jnp.transpose(input_ids.astype(jnp.int32))
    tt_t = jnp.transpose(token_type_ids.astype(jnp.int32))
    out = _bert_emb_sc(ids_t, tt_t, word_emb, pos_emb, type_emb,
                       ln_gamma, ln_beta)
    return out.reshape(B, S, D)


# speedup vs baseline: 3.6210x; 3.6210x over previous
import functools

import jax
import jax.numpy as jnp
from jax import lax
from jax.experimental import pallas as pl
from jax.experimental.pallas import tpu as pltpu
from jax.experimental.pallas import tpu_sc as plsc

B = 1024
S = 512
D = 256
L = 16
NC = 2
NS = 16
NW = NC * NS
SPW = S // NW
NB = 64
CPB = B // NB
NCH = SPW * CPB
NBUF = 4
DJ = D // L
EPS = 1e-12


def _full16(x, dtype=jnp.int32):
    return lax.broadcast_in_dim(jnp.asarray(x, dtype=dtype), (L,), ())


_GDN = lax.GatherDimensionNumbers(
    offset_dims=(), collapsed_slice_dims=(0,), start_index_map=(0,))


def _lane_splat(vec, lane):
    idx = _full16(lane)[:, None]
    return lax.gather(vec, idx, _GDN, (1,),
                      mode=lax.GatherScatterMode.PROMISE_IN_BOUNDS)


def _rsqrt16(v):
    iv = plsc.bitcast(v, jnp.int32)
    magic = _full16(0x5F3759DF)
    y = plsc.bitcast(magic - (iv >> 1), jnp.float32)
    half = v * jnp.float32(0.5)
    for _ in range(3):
        y = y * (jnp.float32(1.5) - half * y * y)
    return y


def _sc_body(ids_h, tt_h, word_h, pos_h, type_h, gam_h, bet_h, out_h,
             idx_v, ttv_v, pos_v, typ_v, gam_v, bet_v, pt0_v, rows_v,
             g0, g1, g2, g3, s0, s1, s2, s3):
    gsems = [g0, g1, g2, g3]
    ssems = [s0, s1, s2, s3]
    wid = lax.axis_index("s") * NC + lax.axis_index("c")
    p0 = wid * SPW

    pltpu.sync_copy(ids_h.at[pl.ds(p0, SPW)], idx_v)
    pltpu.sync_copy(tt_h.at[pl.ds(p0, SPW)], ttv_v)
    pltpu.sync_copy(pos_h.at[pl.ds(p0, SPW)], pos_v)
    pltpu.sync_copy(type_h, typ_v)
    pltpu.sync_copy(gam_h, gam_v)
    pltpu.sync_copy(bet_h, bet_v)

    t0 = [typ_v[0, pl.ds(L * j, L)] for j in range(DJ)]
    dt = [typ_v[1, pl.ds(L * j, L)] - t0[j] for j in range(DJ)]
    gam = [gam_v[pl.ds(L * j, L)] for j in range(DJ)]
    bet = [bet_v[pl.ds(L * j, L)] for j in range(DJ)]

    for p in range(SPW):
        for j in range(DJ):
            pt0_v[p, pl.ds(L * j, L)] = pos_v[p, pl.ds(L * j, L)] + t0[j]

    inv_d = jnp.float32(1.0 / D)

    def chunk_coords(gg):
        s_local = gg // CPB
        b0 = (gg % CPB) * NB
        return s_local, b0

    def gather_copy(gg, buf):
        s_local, b0 = chunk_coords(gg)
        return pltpu.make_async_copy(
            word_h.at[idx_v.at[s_local, pl.ds(b0, NB)]],
            rows_v.at[buf], gsems[buf])

    def scatter_copy(gg, buf):
        s_local, b0 = chunk_coords(gg)
        col0 = (p0 + s_local) * D
        return pltpu.make_async_copy(
            rows_v.at[buf], out_h.at[pl.ds(b0, NB), pl.ds(col0, D)],
            ssems[buf])

    gather_copy(0, 0).start()
    gather_copy(1, 1).start()

    @pl.loop(0, NCH // NBUF)
    def _outer(g):
        for ph in range(NBUF):
            gg = g * NBUF + ph

            @pl.when(gg >= 2)
            def _():
                scatter_copy(gg - 2, (ph + 2) % NBUF).wait()

            @pl.when(gg + 2 < NCH)
            def _():
                gather_copy(gg + 2, (ph + 2) % NBUF).start()

            gather_copy(gg, ph).wait()

            s_local, b0 = chunk_coords(gg)
            prow = [pt0_v[s_local, pl.ds(L * j, L)] for j in range(DJ)]
            rbuf = rows_v.at[ph]

            @pl.loop(0, NB)
            def _tok(t):
                tbase = (t // L) * L
                tt16 = ttv_v[s_local, pl.ds(b0 + tbase, L)]
                ttf = _lane_splat(tt16, t - tbase).astype(jnp.float32)
                x = []
                for j in range(DJ):
                    w = rbuf[t, pl.ds(L * j, L)]
                    x.append(w + prow[j] + ttf * dt[j])
                acc = x[0]
                for j in range(1, DJ):
                    acc = acc + x[j]
                acc2 = x[0] * x[0]
                for j in range(1, DJ):
                    acc2 = acc2 + x[j] * x[j]
                mu = jnp.sum(acc) * inv_d
                m2 = jnp.sum(acc2) * inv_d
                var = m2 - mu * mu
                r = _rsqrt16(_full16(var + jnp.float32(EPS), jnp.float32))
                muv = _full16(mu, jnp.float32)
                for j in range(DJ):
                    rbuf[t, pl.ds(L * j, L)] = (
                        (x[j] - muv) * r * gam[j] + bet[j])

            scatter_copy(gg, ph).start()

    scatter_copy(NCH - 2, (NCH - 2) % NBUF).wait()
    scatter_copy(NCH - 1, (NCH - 1) % NBUF).wait()


@functools.partial(
    pl.kernel,
    out_type=jax.ShapeDtypeStruct((B, S * D), jnp.float32),
    mesh=plsc.VectorSubcoreMesh(
        core_axis_name="c", subcore_axis_name="s",
        num_cores=NC, num_subcores=NS),
    compiler_params=pltpu.CompilerParams(needs_layout_passes=False),
    scratch_types=[
        pltpu.VMEM((SPW, B), jnp.int32),
        pltpu.VMEM((SPW, B), jnp.int32),
        pltpu.VMEM((SPW, D), jnp.float32),
        pltpu.VMEM((2, D), jnp.float32),
        pltpu.VMEM((D,), jnp.float32),
        pltpu.VMEM((D,), jnp.float32),
        pltpu.VMEM((SPW, D), jnp.float32),
        pltpu.VMEM((NBUF, NB, D), jnp.float32),
        pltpu.SemaphoreType.DMA,
        pltpu.SemaphoreType.DMA,
        pltpu.SemaphoreType.DMA,
        pltpu.SemaphoreType.DMA,
        pltpu.SemaphoreType.DMA,
        pltpu.SemaphoreType.DMA,
        pltpu.SemaphoreType.DMA,
        pltpu.SemaphoreType.DMA,
    ],
)
def _bert_emb_sc(ids_h, tt_h, word_h, pos_h, type_h, gam_h, bet_h, out_h,
                 *scratch):
    _sc_body(ids_h, tt_h, word_h, pos_h, type_h, gam_h, bet_h, out_h,
             *scratch)


def kernel(input_ids, token_type_ids, word_emb, pos_emb, type_emb,
           ln_gamma, ln_beta):
    ids_t = jnp.transpose(input_ids.astype(jnp.int32))
    tt_t = jnp.transpose(token_type_ids.astype(jnp.int32))
    out = _bert_emb_sc(ids_t, tt_t, word_emb, pos_emb, type_emb,
                       ln_gamma, ln_beta)
    return out.reshape(B, S, D)


# speedup vs baseline: 4.7692x; 1.3171x over previous
import functools

import jax
import jax.numpy as jnp
from jax import lax
from jax.experimental import pallas as pl
from jax.experimental.pallas import tpu as pltpu
from jax.experimental.pallas import tpu_sc as plsc

B = 1024
S = 512
D = 256
L = 16
NC = 2
NS = 16
NW = NC * NS
BPW = B // NW
NB = 64
WPS = S // NB
NCH = WPS * BPW
NBUF = 4
DJ = D // L
EPS = 1e-12


def _full16(x, dtype=jnp.int32):
    return lax.broadcast_in_dim(jnp.asarray(x, dtype=dtype), (L,), ())


def _rsqrt16(v):
    iv = plsc.bitcast(v, jnp.int32)
    magic = _full16(0x5F3759DF)
    y = plsc.bitcast(magic - (iv >> 1), jnp.float32)
    half = v * jnp.float32(0.5)
    for _ in range(3):
        y = y * (jnp.float32(1.5) - half * y * y)
    return y


def _sc_body(ids_h, tt_h, word_h, pos_h, type_h, gam_h, bet_h, out_h,
             idx_v, ttv_v, pos_v, typ_v, gam_v, bet_v, rows_v,
             g0, g1, g2, g3, s0_, s1_, s2_, s3_):
    gsems = [g0, g1, g2, g3]
    ssems = [s0_, s1_, s2_, s3_]
    wid = lax.axis_index("s") * NC + lax.axis_index("c")
    b_base = wid * BPW

    pltpu.sync_copy(ids_h.at[pl.ds(b_base, BPW)], idx_v)
    pltpu.sync_copy(tt_h.at[pl.ds(b_base, BPW)], ttv_v)
    pltpu.sync_copy(type_h, typ_v)
    pltpu.sync_copy(gam_h, gam_v)
    pltpu.sync_copy(bet_h, bet_v)

    t0 = [typ_v[0, pl.ds(L * j, L)] for j in range(DJ)]
    dt = [typ_v[1, pl.ds(L * j, L)] - t0[j] for j in range(DJ)]
    gam = [gam_v[pl.ds(L * j, L)] for j in range(DJ)]
    bet = [bet_v[pl.ds(L * j, L)] for j in range(DJ)]

    inv_d = jnp.float32(1.0 / D)

    def chunk_coords(gg):
        w0 = gg // BPW
        b_local = gg % BPW
        return w0 * NB, b_local

    def gather_copy(gg, buf):
        s0, b_local = chunk_coords(gg)
        return pltpu.make_async_copy(
            word_h.at[idx_v.at[b_local, pl.ds(s0, NB)]],
            rows_v.at[buf], gsems[buf])

    def scatter_copy(gg, buf):
        s0, b_local = chunk_coords(gg)
        return pltpu.make_async_copy(
            rows_v.at[buf], out_h.at[b_base + b_local, pl.ds(s0, NB)],
            ssems[buf])

    def stage_pos_window(s0):
        pltpu.sync_copy(pos_h.at[pl.ds(s0, NB)], pos_v)

        @pl.loop(0, NB)
        def _row(t):
            for j in range(DJ):
                pos_v[t, pl.ds(L * j, L)] = (
                    pos_v[t, pl.ds(L * j, L)] + t0[j])

    stage_pos_window(0)

    gather_copy(0, 0).start()
    gather_copy(1, 1).start()

    @pl.loop(0, NCH // NBUF)
    def _outer(g):
        for ph in range(NBUF):
            gg = g * NBUF + ph
            s0, b_local = chunk_coords(gg)

            @pl.when(jnp.logical_and(b_local == 0, gg > 0))
            def _():
                stage_pos_window(s0)

            @pl.when(gg >= 2)
            def _():
                scatter_copy(gg - 2, (ph + 2) % NBUF).wait()

            @pl.when(gg + 2 < NCH)
            def _():
                gather_copy(gg + 2, (ph + 2) % NBUF).start()

            gather_copy(gg, ph).wait()
            rbuf = rows_v.at[ph]

            @pl.loop(0, NB, unroll=4)
            def _tok(t):
                tsp = plsc.load_gather(
                    ttv_v, [_full16(b_local), _full16(s0 + t)])
                ttf = tsp.astype(jnp.float32)
                x = []
                for j in range(DJ):
                    w = rbuf[t, pl.ds(L * j, L)]
                    x.append(w + pos_v[t, pl.ds(L * j, L)] + ttf * dt[j])
                acc = x[0]
                for j in range(1, DJ):
                    acc = acc + x[j]
                acc2 = x[0] * x[0]
                for j in range(1, DJ):
                    acc2 = acc2 + x[j] * x[j]
                mu = jnp.sum(acc) * inv_d
                m2 = jnp.sum(acc2) * inv_d
                var = m2 - mu * mu
                r = _rsqrt16(_full16(var + jnp.float32(EPS), jnp.float32))
                muv = _full16(mu, jnp.float32)
                for j in range(DJ):
                    rbuf[t, pl.ds(L * j, L)] = (
                        (x[j] - muv) * r * gam[j] + bet[j])

            scatter_copy(gg, ph).start()

    scatter_copy(NCH - 2, (NCH - 2) % NBUF).wait()
    scatter_copy(NCH - 1, (NCH - 1) % NBUF).wait()


@functools.partial(
    pl.kernel,
    out_type=jax.ShapeDtypeStruct((B, S, D), jnp.float32),
    mesh=plsc.VectorSubcoreMesh(
        core_axis_name="c", subcore_axis_name="s",
        num_cores=NC, num_subcores=NS),
    compiler_params=pltpu.CompilerParams(needs_layout_passes=False),
    scratch_types=[
        pltpu.VMEM((BPW, S), jnp.int32),
        pltpu.VMEM((BPW, S), jnp.int32),
        pltpu.VMEM((NB, D), jnp.float32),
        pltpu.VMEM((2, D), jnp.float32),
        pltpu.VMEM((D,), jnp.float32),
        pltpu.VMEM((D,), jnp.float32),
        pltpu.VMEM((NBUF, NB, D), jnp.float32),
        pltpu.SemaphoreType.DMA,
        pltpu.SemaphoreType.DMA,
        pltpu.SemaphoreType.DMA,
        pltpu.SemaphoreType.DMA,
        pltpu.SemaphoreType.DMA,
        pltpu.SemaphoreType.DMA,
        pltpu.SemaphoreType.DMA,
        pltpu.SemaphoreType.DMA,
    ],
)
def _bert_emb_sc(ids_h, tt_h, word_h, pos_h, type_h, gam_h, bet_h, out_h,
                 *scratch):
    _sc_body(ids_h, tt_h, word_h, pos_h, type_h, gam_h, bet_h, out_h,
             *scratch)


def kernel(input_ids, token_type_ids, word_emb, pos_emb, type_emb,
           ln_gamma, ln_beta):
    return _bert_emb_sc(input_ids.astype(jnp.int32),
                        token_type_ids.astype(jnp.int32),
                        word_emb, pos_emb, type_emb, ln_gamma, ln_beta)
